# baseline (speedup 1.0000x reference)
import jax
import jax.numpy as jnp
from jax.experimental import pallas as pl
from jax.experimental.pallas import tpu as pltpu

S = 7
C = 20
NBOX = 2
CELL = C + NBOX * 5
BATCH = 8
MID = 2048
IN_F = 1024 * S * S
OUT_F = S * S * CELL
K_BLK = 1024
K_TILES = IN_F // K_BLK


def _head_kernel(x_ref, w1_ref, b1_ref, w2_ref, b2_ref, out_ref, acc_ref):
    k = pl.program_id(0)

    @pl.when(k == 0)
    def _init():
        acc_ref[...] = jnp.broadcast_to(b1_ref[...], acc_ref.shape)

    acc_ref[...] += jnp.dot(
        x_ref[...], w1_ref[...], preferred_element_type=jnp.float32
    )

    @pl.when(k == K_TILES - 1)
    def _finish():
        h = acc_ref[...]
        h = jnp.where(h > 0, h, 0.1 * h)
        o = jnp.dot(h, w2_ref[...], preferred_element_type=jnp.float32)
        o = o + b2_ref[...]
        col = jax.lax.broadcasted_iota(jnp.int32, o.shape, 1)
        r = col % CELL
        m = (r < C) & ((r % 5 == 1) | (r % 5 == 2))
        out_ref[...] = jnp.where(m, jax.nn.sigmoid(o), o)


def kernel(x, W1, b1, W2, b2):
    x2 = x.reshape(BATCH, IN_F)
    out = pl.pallas_call(
        _head_kernel,
        grid=(K_TILES,),
        in_specs=[
            pl.BlockSpec((BATCH, K_BLK), lambda k: (0, k)),
            pl.BlockSpec((K_BLK, MID), lambda k: (k, 0)),
            pl.BlockSpec((1, MID), lambda k: (0, 0)),
            pl.BlockSpec((MID, OUT_F), lambda k: (0, 0)),
            pl.BlockSpec((1, OUT_F), lambda k: (0, 0)),
        ],
        out_specs=pl.BlockSpec((BATCH, OUT_F), lambda k: (0, 0)),
        out_shape=jax.ShapeDtypeStruct((BATCH, OUT_F), jnp.float32),
        scratch_shapes=[pltpu.VMEM((BATCH, MID), jnp.float32)],
        compiler_params=pltpu.CompilerParams(
            dimension_semantics=("arbitrary",),
        ),
    )(x2, W1, b1[None, :], W2, b2[None, :])
    return out.reshape(-1, S, S, CELL)
